# baseline (speedup 1.0000x reference)
import jax
import jax.numpy as jnp
from jax import lax
from jax.experimental import pallas as pl
from jax.experimental.pallas import tpu as pltpu
from jax.experimental.pallas import tpu_sc as plsc

V = 100000
D = 300
B = 4096
H = 50

NC = 2
NS = 16
L = 16
NW = NC * NS
ROWS = 2 * H
SCALE = 1.0 / H

G = V * D // L
GPR = 20
NG = ROWS * GPR
GPD = 80
NDMA = NG // GPD

CHUNK_OFFS = tuple(c * L for c in range(D // L)) + (D - L,)


def _build(batch):
    bags_per_w = batch // NW
    pairs_per_w = bags_per_w // 2

    def build_gids(idx_ref, gid_ref, ph_ref):
        it = lax.iota(jnp.int32, L)

        def rows(ivec, r0, lanes):
            ph_ref[pl.ds(r0, L)] = (ivec * 12) & 15
            for t in lanes:
                g0 = (ivec[t] * 75) >> 2
                va = jnp.minimum(g0 + it, G - 1)
                vb = jnp.minimum(g0 + L + it, G - 1)
                base = (r0 + t) * GPR
                gid_ref[pl.ds(base, L)] = va
                gid_ref[pl.ds(base + L, L)] = vb

        for r0 in (0, 16, 32, 48, 64, 80):
            rows(idx_ref[pl.ds(r0, L)], r0, range(L))
        rows(idx_ref[pl.ds(ROWS - L, L)], ROWS - L, (12, 13, 14, 15))

    def acc_pair(ph_ref, buf_ref, out_ref, bag0):
        zero = jnp.zeros((L,), jnp.float32)
        it = lax.iota(jnp.int32, L)

        def rows(accs, pvec, r0, lanes):
            for t in lanes:
                p = pvec[t]
                pv0 = p + it
                col0 = pv0 & 15
                ro0 = pv0 >> 4
                pv1 = p + 12 + it
                col1 = pv1 & 15
                ro1 = (pv1 >> 4) + 17
                rbase = (r0 + t) * GPR
                new = [accs[c] + plsc.load_gather(
                           buf_ref, (ro0 + (rbase + c), col0))
                       for c in range(18)]
                new.append(accs[18] + plsc.load_gather(
                    buf_ref, (ro1 + rbase, col1)))
                accs = tuple(new)
            return accs

        for half in range(2):
            hb = half * H

            def grp(q, accs, _hb=hb):
                r0 = _hb + 16 * q
                return rows(accs, ph_ref[pl.ds(r0, L)], r0, range(L))

            accs = lax.fori_loop(0, 3, grp, tuple(zero for _ in CHUNK_OFFS))
            accs = rows(accs, ph_ref[pl.ds(hb + H - L, L)],
                        hb + H - L, (14, 15))
            for a, off in zip(accs, CHUNK_OFFS):
                out_ref[bag0 + half, pl.ds(off, L)] = a * SCALE

    def body(tg_hbm, x2_hbm, out_hbm, idx_a, idx_b, gid_a, gid_b,
             ph_a, ph_b, buf_a, buf_b, out_v, sem_a, sem_b, isem_a, isem_b):
        wid = lax.axis_index("s") * NC + lax.axis_index("c")
        jbase = wid * pairs_per_w
        last = pairs_per_w - 1

        def istart(idx, isem, j):
            jg = jbase + jnp.minimum(j, last)
            pltpu.make_async_copy(x2_hbm.at[jg], idx, isem).start()

        def iwait(idx, isem):
            pltpu.make_async_copy(x2_hbm.at[jbase], idx, isem).wait()

        def gstart(gid, buf, sem):
            for d in range(NDMA):
                pltpu.make_async_copy(tg_hbm.at[gid.at[pl.ds(GPD * d, GPD)]],
                                      buf.at[pl.ds(GPD * d, GPD)], sem).start()

        def gwait(gid, buf, sem):
            for _ in range(NDMA):
                pltpu.make_async_copy(tg_hbm.at[gid.at[pl.ds(0, GPD)]],
                                      buf.at[pl.ds(0, GPD)], sem).wait()

        istart(idx_a, isem_a, 0)
        istart(idx_b, isem_b, 1)
        iwait(idx_a, isem_a)
        build_gids(idx_a, gid_a, ph_a)
        gstart(gid_a, buf_a, sem_a)
        iwait(idx_b, isem_b)
        build_gids(idx_b, gid_b, ph_b)
        gstart(gid_b, buf_b, sem_b)

        def outer(g, carry):
            gwait(gid_a, buf_a, sem_a)
            istart(idx_a, isem_a, 2 * g + 2)
            acc_pair(ph_a, buf_a, out_v, 4 * g)
            iwait(idx_a, isem_a)
            build_gids(idx_a, gid_a, ph_a)
            gstart(gid_a, buf_a, sem_a)
            gwait(gid_b, buf_b, sem_b)
            istart(idx_b, isem_b, 2 * g + 3)
            acc_pair(ph_b, buf_b, out_v, 4 * g + 2)
            iwait(idx_b, isem_b)
            build_gids(idx_b, gid_b, ph_b)
            gstart(gid_b, buf_b, sem_b)
            return carry

        lax.fori_loop(0, pairs_per_w // 2, outer, 0)
        gwait(gid_a, buf_a, sem_a)
        gwait(gid_b, buf_b, sem_b)
        pltpu.sync_copy(out_v, out_hbm.at[pl.ds(wid * bags_per_w, bags_per_w)])

    out_type = jax.ShapeDtypeStruct((batch, D), jnp.float32)
    scratch_types = [
        pltpu.VMEM((ROWS,), jnp.int32),
        pltpu.VMEM((ROWS,), jnp.int32),
        pltpu.VMEM((NG + L,), jnp.int32),
        pltpu.VMEM((NG + L,), jnp.int32),
        pltpu.VMEM((ROWS,), jnp.int32),
        pltpu.VMEM((ROWS,), jnp.int32),
        pltpu.VMEM((NG, L), jnp.float32),
        pltpu.VMEM((NG, L), jnp.float32),
        pltpu.VMEM((bags_per_w, D), jnp.float32),
        pltpu.SemaphoreType.DMA,
        pltpu.SemaphoreType.DMA,
        pltpu.SemaphoreType.DMA,
        pltpu.SemaphoreType.DMA,
    ]
    return body, out_type, scratch_types


_body, _out_type, _scratch_types = _build(B)
_embed_mean = pl.kernel(
    _body,
    out_type=_out_type,
    mesh=plsc.VectorSubcoreMesh(core_axis_name="c", subcore_axis_name="s"),
    scratch_types=_scratch_types,
    compiler_params=pltpu.CompilerParams(use_tc_tiling_on_sc=False,
                                         needs_layout_passes=False),
)


def kernel(x, table):
    tg = table.reshape(G, L)
    x2 = x.reshape(B // 2, ROWS)
    return _embed_mean(tg, x2)


# speedup vs baseline: 1.0040x; 1.0040x over previous
import jax
import jax.numpy as jnp
from jax import lax
from jax.experimental import pallas as pl
from jax.experimental.pallas import tpu as pltpu
from jax.experimental.pallas import tpu_sc as plsc

V = 100000
D = 300
B = 4096
H = 50

NC = 2
NS = 16
L = 16
NW = NC * NS
ROWS = 2 * H
SCALE = 1.0 / H

G = V * D // L
GPR = 20
NG = ROWS * GPR
GPD = 2000
NDMA = NG // GPD

CHUNK_OFFS = tuple(c * L for c in range(D // L)) + (D - L,)


def _build(batch):
    bags_per_w = batch // NW
    pairs_per_w = bags_per_w // 2

    def build_gids(idx_ref, gid_ref, ph_ref):
        it = lax.iota(jnp.int32, L)

        def rows(ivec, r0, lanes):
            ph_ref[pl.ds(r0, L)] = (ivec * 12) & 15
            for t in lanes:
                g0 = (ivec[t] * 75) >> 2
                va = jnp.minimum(g0 + it, G - 1)
                vb = jnp.minimum(g0 + L + it, G - 1)
                base = (r0 + t) * GPR
                gid_ref[pl.ds(base, L)] = va
                gid_ref[pl.ds(base + L, L)] = vb

        for r0 in (0, 16, 32, 48, 64, 80):
            rows(idx_ref[pl.ds(r0, L)], r0, range(L))
        rows(idx_ref[pl.ds(ROWS - L, L)], ROWS - L, (12, 13, 14, 15))

    def acc_pair(ph_ref, buf_ref, out_ref, bag0):
        zero = jnp.zeros((L,), jnp.float32)
        it = lax.iota(jnp.int32, L)

        def rows(accs, pvec, r0, lanes):
            for t in lanes:
                p = pvec[t]
                pv0 = p + it
                col0 = pv0 & 15
                ro0 = pv0 >> 4
                pv1 = p + 12 + it
                col1 = pv1 & 15
                ro1 = (pv1 >> 4) + 17
                rbase = (r0 + t) * GPR
                new = [accs[c] + plsc.load_gather(
                           buf_ref, (ro0 + (rbase + c), col0))
                       for c in range(18)]
                new.append(accs[18] + plsc.load_gather(
                    buf_ref, (ro1 + rbase, col1)))
                accs = tuple(new)
            return accs

        for half in range(2):
            hb = half * H

            def grp(q, accs, _hb=hb):
                r0 = _hb + 16 * q
                return rows(accs, ph_ref[pl.ds(r0, L)], r0, range(L))

            accs = lax.fori_loop(0, 3, grp, tuple(zero for _ in CHUNK_OFFS))
            accs = rows(accs, ph_ref[pl.ds(hb + H - L, L)],
                        hb + H - L, (14, 15))
            for a, off in zip(accs, CHUNK_OFFS):
                out_ref[bag0 + half, pl.ds(off, L)] = a * SCALE

    def body(tg_hbm, x2_hbm, out_hbm, idx_a, idx_b, gid_a, gid_b,
             ph_a, ph_b, buf_a, buf_b, out_v, sem_a, sem_b, isem_a, isem_b):
        wid = lax.axis_index("s") * NC + lax.axis_index("c")
        jbase = wid * pairs_per_w
        last = pairs_per_w - 1

        def istart(idx, isem, j):
            jg = jbase + jnp.minimum(j, last)
            pltpu.make_async_copy(x2_hbm.at[jg], idx, isem).start()

        def iwait(idx, isem):
            pltpu.make_async_copy(x2_hbm.at[jbase], idx, isem).wait()

        def gstart(gid, buf, sem):
            for d in range(NDMA):
                pltpu.make_async_copy(tg_hbm.at[gid.at[pl.ds(GPD * d, GPD)]],
                                      buf.at[pl.ds(GPD * d, GPD)], sem).start()

        def gwait(gid, buf, sem):
            for _ in range(NDMA):
                pltpu.make_async_copy(tg_hbm.at[gid.at[pl.ds(0, GPD)]],
                                      buf.at[pl.ds(0, GPD)], sem).wait()

        istart(idx_a, isem_a, 0)
        istart(idx_b, isem_b, 1)
        iwait(idx_a, isem_a)
        build_gids(idx_a, gid_a, ph_a)
        gstart(gid_a, buf_a, sem_a)
        iwait(idx_b, isem_b)
        build_gids(idx_b, gid_b, ph_b)
        gstart(gid_b, buf_b, sem_b)

        def outer(g, carry):
            gwait(gid_a, buf_a, sem_a)
            istart(idx_a, isem_a, 2 * g + 2)
            acc_pair(ph_a, buf_a, out_v, 4 * g)
            iwait(idx_a, isem_a)
            build_gids(idx_a, gid_a, ph_a)
            gstart(gid_a, buf_a, sem_a)
            gwait(gid_b, buf_b, sem_b)
            istart(idx_b, isem_b, 2 * g + 3)
            acc_pair(ph_b, buf_b, out_v, 4 * g + 2)
            iwait(idx_b, isem_b)
            build_gids(idx_b, gid_b, ph_b)
            gstart(gid_b, buf_b, sem_b)
            return carry

        lax.fori_loop(0, pairs_per_w // 2, outer, 0)
        gwait(gid_a, buf_a, sem_a)
        gwait(gid_b, buf_b, sem_b)
        pltpu.sync_copy(out_v, out_hbm.at[pl.ds(wid * bags_per_w, bags_per_w)])

    out_type = jax.ShapeDtypeStruct((batch, D), jnp.float32)
    scratch_types = [
        pltpu.VMEM((ROWS,), jnp.int32),
        pltpu.VMEM((ROWS,), jnp.int32),
        pltpu.VMEM((NG + L,), jnp.int32),
        pltpu.VMEM((NG + L,), jnp.int32),
        pltpu.VMEM((ROWS,), jnp.int32),
        pltpu.VMEM((ROWS,), jnp.int32),
        pltpu.VMEM((NG, L), jnp.float32),
        pltpu.VMEM((NG, L), jnp.float32),
        pltpu.VMEM((bags_per_w, D), jnp.float32),
        pltpu.SemaphoreType.DMA,
        pltpu.SemaphoreType.DMA,
        pltpu.SemaphoreType.DMA,
        pltpu.SemaphoreType.DMA,
    ]
    return body, out_type, scratch_types


_body, _out_type, _scratch_types = _build(B)
_embed_mean = pl.kernel(
    _body,
    out_type=_out_type,
    mesh=plsc.VectorSubcoreMesh(core_axis_name="c", subcore_axis_name="s"),
    scratch_types=_scratch_types,
    compiler_params=pltpu.CompilerParams(use_tc_tiling_on_sc=False,
                                         needs_layout_passes=False),
)


def kernel(x, table):
    tg = table.reshape(G, L)
    x2 = x.reshape(B // 2, ROWS)
    return _embed_mean(tg, x2)


# speedup vs baseline: 2.4900x; 2.4801x over previous
import jax
import jax.numpy as jnp
from jax import lax
from jax.experimental import pallas as pl
from jax.experimental.pallas import tpu as pltpu
from jax.experimental.pallas import tpu_sc as plsc

V = 100000
D = 300
B = 4096
H = 50

NC = 2
NS = 16
L = 16
NW = NC * NS
ROWS = 2 * H
SCALE = 1.0 / H

CHUNK_OFFS = tuple(c * L for c in range(D // L)) + (D - L,)


def _build(batch):
    bags_per_w = batch // NW
    pairs_per_w = bags_per_w // 2

    def acc_pair(rows_ref, out_ref, bag0):
        zero = jnp.zeros((L,), jnp.float32)
        for half in range(2):
            def bodyf(r, accs, _half=half):
                row = _half * H + r
                return tuple(a + rows_ref[row, pl.ds(off, L)]
                             for a, off in zip(accs, CHUNK_OFFS))
            accs = lax.fori_loop(0, H, bodyf, tuple(zero for _ in CHUNK_OFFS))
            for a, off in zip(accs, CHUNK_OFFS):
                out_ref[bag0 + half, pl.ds(off, L)] = a * SCALE

    def body(table_hbm, x2_hbm, out_hbm, idx_a, idx_b,
             buf_a, buf_b, out_v, sem_a, sem_b, isem_a, isem_b):
        wid = lax.axis_index("s") * NC + lax.axis_index("c")
        jbase = wid * pairs_per_w
        last = pairs_per_w - 1

        def istart(idx, isem, j):
            jg = jbase + jnp.minimum(j, last)
            pltpu.make_async_copy(x2_hbm.at[jg], idx, isem).start()

        def iwait(idx, isem):
            pltpu.make_async_copy(x2_hbm.at[jbase], idx, isem).wait()

        def row_starts(ivec, r0, lanes, buf, sem):
            for t in lanes:
                i = ivec[t]
                pltpu.make_async_copy(table_hbm.at[i],
                                      buf.at[r0 + t], sem).start()

        def gstart(idx, buf, sem):
            for r0 in (0, 16, 32, 48, 64, 80):
                ivec = idx[pl.ds(r0, L)]
                row_starts(ivec, r0, range(L), buf, sem)
            ivec = idx[pl.ds(ROWS - L, L)]
            row_starts(ivec, ROWS - L, (12, 13, 14, 15), buf, sem)

        def gwait(buf, sem):
            for _ in range(ROWS):
                pltpu.make_async_copy(table_hbm.at[0], buf.at[0], sem).wait()

        istart(idx_a, isem_a, 0)
        istart(idx_b, isem_b, 1)
        iwait(idx_a, isem_a)
        gstart(idx_a, buf_a, sem_a)
        iwait(idx_b, isem_b)
        gstart(idx_b, buf_b, sem_b)

        def outer(g, carry):
            gwait(buf_a, sem_a)
            istart(idx_a, isem_a, 2 * g + 2)
            acc_pair(buf_a, out_v, 4 * g)
            iwait(idx_a, isem_a)
            gstart(idx_a, buf_a, sem_a)
            gwait(buf_b, sem_b)
            istart(idx_b, isem_b, 2 * g + 3)
            acc_pair(buf_b, out_v, 4 * g + 2)
            iwait(idx_b, isem_b)
            gstart(idx_b, buf_b, sem_b)
            return carry

        lax.fori_loop(0, pairs_per_w // 2, outer, 0)
        gwait(buf_a, sem_a)
        gwait(buf_b, sem_b)
        pltpu.sync_copy(out_v, out_hbm.at[pl.ds(wid * bags_per_w, bags_per_w)])

    out_type = jax.ShapeDtypeStruct((batch, D), jnp.float32)
    scratch_types = [
        pltpu.VMEM((ROWS,), jnp.int32),
        pltpu.VMEM((ROWS,), jnp.int32),
        pltpu.VMEM((ROWS, D), jnp.float32),
        pltpu.VMEM((ROWS, D), jnp.float32),
        pltpu.VMEM((bags_per_w, D), jnp.float32),
        pltpu.SemaphoreType.DMA,
        pltpu.SemaphoreType.DMA,
        pltpu.SemaphoreType.DMA,
        pltpu.SemaphoreType.DMA,
    ]
    return body, out_type, scratch_types


_body, _out_type, _scratch_types = _build(B)
_embed_mean = pl.kernel(
    _body,
    out_type=_out_type,
    mesh=plsc.VectorSubcoreMesh(core_axis_name="c", subcore_axis_name="s"),
    scratch_types=_scratch_types,
    compiler_params=pltpu.CompilerParams(use_tc_tiling_on_sc=True,
                                         needs_layout_passes=False),
)


def kernel(x, table):
    x2 = x.reshape(B // 2, ROWS)
    return _embed_mean(table, x2)
